# baseline (speedup 1.0000x reference)
import functools
import jax
import jax.numpy as jnp
from jax import lax
from jax.experimental import pallas as pl
from jax.experimental.pallas import tpu as pltpu
from jax.experimental.pallas import tpu_sc as plsc

BATCH, SEQ, DIM = 4096, 200, 64
K = SEQ * DIM
NW = 32
NCHUNK = 3200
CPW = NCHUNK // NW
KPW = K // NW
NBUF = 5
NGRP = CPW // NBUF


def _sc_body(x_hbm, pos_hbm, out_hbm, pv, bufs, *sems):
    gsem = sems[:NBUF]
    ssem = sems[NBUF:]
    wid = lax.axis_index("s") * 2 + lax.axis_index("c")
    base_m = wid * CPW
    base_k = wid * KPW

    pltpu.sync_copy(pos_hbm.at[pl.ds(base_k, KPW)], pv)

    def gather(m, p):
        return pltpu.make_async_copy(x_hbm.at[m], bufs.at[p], gsem[p])

    def scatter(m, p):
        return pltpu.make_async_copy(bufs.at[p], out_hbm.at[m], ssem[p])

    for p in range(NBUF):
        gather(base_m + p, p).start()

    def gbody(g, carry):
        for p in range(NBUF):
            m = base_m + g * NBUF + p
            gather(m, p).wait()
            k0 = 8 * lax.shift_right_logical(m, 1) - base_k
            buf = bufs.at[p]
            for i in range(8):
                idx = jnp.full((16,), k0 + i, dtype=jnp.int32)
                splat = plsc.load_gather(pv, [idx])
                for c in range(16):
                    for t in range(8):
                        sl = pl.ds(t * 16, 16)
                        buf[c, i, sl] = buf[c, i, sl] + splat
            scatter(m, p).start()

        for p in range(NBUF):
            m = base_m + g * NBUF + p
            scatter(m, p).wait()

            @pl.when(g < NGRP - 1)
            def _(m=m, p=p):
                gather(m + NBUF, p).start()

        return carry

    lax.fori_loop(0, NGRP, gbody, 0)


def kernel(x, pos_table):
    b, s, d = x.shape
    k = s * d
    xt = jnp.transpose(x, (1, 2, 0)).reshape(k, b)
    x4 = xt.reshape(k // 8, 8, b // 128, 128).transpose(0, 2, 1, 3)
    x6 = x4.reshape(NCHUNK, 16, 8, 128)
    posf = pos_table.reshape(k)

    mesh = plsc.VectorSubcoreMesh(core_axis_name="c", subcore_axis_name="s")
    f = functools.partial(
        pl.kernel,
        mesh=mesh,
        out_type=jax.ShapeDtypeStruct((NCHUNK, 16, 8, 128), jnp.float32),
        scratch_types=[
            pltpu.VMEM((KPW,), jnp.float32),
            pltpu.VMEM((NBUF, 16, 8, 128), jnp.float32),
        ]
        + [pltpu.SemaphoreType.DMA] * (2 * NBUF),
        compiler_params=pltpu.CompilerParams(needs_layout_passes=False),
    )(_sc_body)
    out6 = f(x6, posf)

    out_xt = out6.reshape(k // 8, b // 128, 8, 128).transpose(0, 2, 1, 3).reshape(k, b)
    return jnp.transpose(out_xt.reshape(s, d, b), (2, 0, 1))


# speedup vs baseline: 1.1077x; 1.1077x over previous
import functools
import jax
import jax.numpy as jnp
from jax import lax
from jax.experimental import pallas as pl
from jax.experimental.pallas import tpu as pltpu
from jax.experimental.pallas import tpu_sc as plsc

BATCH, SEQ, DIM = 4096, 200, 64
K = SEQ * DIM
S1 = 120
K1 = S1 * DIM
K2 = K - K1
ROW_BLOCK = 256
LANES = 128

NW = 32
NCHUNK = K // 2
M1 = K1 // 2
NCHUNK2 = K2 // 2
CPW = NCHUNK2 // NW
KPW = K2 // NW
NBUF = 5
NGRP = CPW // NBUF


def _tc_body(x_ref, pos_ref, out_ref):
    pv = pos_ref[...]
    for g in range(x_ref.shape[1] // LANES):
        sl = pl.ds(g * LANES, LANES)
        out_ref[:, sl] = x_ref[:, sl] + pv


def _sc_body(x_hbm, pos_hbm, out_hbm, pv, ibufs, obufs, *sems):
    gsem = sems[:NBUF]
    ssem = sems[NBUF:]
    wid = lax.axis_index("s") * 2 + lax.axis_index("c")
    base_m = M1 + wid * CPW
    base_k = K1 + wid * KPW

    pltpu.sync_copy(pos_hbm.at[pl.ds(base_k, KPW)], pv)

    def gather(m, p):
        return pltpu.make_async_copy(x_hbm.at[m], ibufs.at[p], gsem[p])

    def scatter(m, p):
        return pltpu.make_async_copy(obufs.at[p], out_hbm.at[m - M1], ssem[p])

    for p in range(NBUF):
        gather(base_m + p, p).start()

    def gbody(g, carry):
        for p in range(NBUF):
            m = base_m + g * NBUF + p
            gather(m, p).wait()

            @pl.when(g > 0)
            def _(m=m, p=p):
                scatter(m - NBUF, p).wait()

            k0 = 8 * lax.shift_right_logical(m, 2) - base_k
            ib = ibufs.at[p]
            ob = obufs.at[p]
            for i in range(8):
                idx = jnp.full((16,), k0 + i, dtype=jnp.int32)
                splat = plsc.load_gather(pv, [idx])
                for c in range(8):
                    for t in range(8):
                        sl = pl.ds(t * 16, 16)
                        ob[c, i, sl] = ib[c, i, sl] + splat
            scatter(m, p).start()

            @pl.when(g < NGRP - 1)
            def _(m=m, p=p):
                gather(m + NBUF, p).start()

        return carry

    lax.fori_loop(0, NGRP, gbody, 0)

    for p in range(NBUF):
        scatter(base_m + (NGRP - 1) * NBUF + p, p).wait()


def kernel(x, pos_table):
    b, s, d = x.shape
    k = s * d
    xt = jnp.transpose(x, (1, 2, 0)).reshape(k, b)
    posf = pos_table.reshape(k)

    posb = jnp.broadcast_to(posf.reshape(k, 1), (k, LANES))
    out_tc = pl.pallas_call(
        _tc_body,
        grid=(K1 // ROW_BLOCK,),
        in_specs=[
            pl.BlockSpec((ROW_BLOCK, b), lambda i: (i, 0)),
            pl.BlockSpec((ROW_BLOCK, LANES), lambda i: (i, 0)),
        ],
        out_specs=pl.BlockSpec((ROW_BLOCK, b), lambda i: (i, 0)),
        out_shape=jax.ShapeDtypeStruct((K1, b), x.dtype),
    )(xt, posb)

    x4 = xt.reshape(k // 8, 8, b // 128, 128).transpose(0, 2, 1, 3)
    x6 = x4.reshape(NCHUNK, 8, 8, 128)
    mesh = plsc.VectorSubcoreMesh(core_axis_name="c", subcore_axis_name="s")
    f = functools.partial(
        pl.kernel,
        mesh=mesh,
        out_type=jax.ShapeDtypeStruct((NCHUNK2, 8, 8, 128), jnp.float32),
        scratch_types=[
            pltpu.VMEM((KPW,), jnp.float32),
            pltpu.VMEM((NBUF, 8, 8, 128), jnp.float32),
            pltpu.VMEM((NBUF, 8, 8, 128), jnp.float32),
        ]
        + [pltpu.SemaphoreType.DMA] * (2 * NBUF),
        compiler_params=pltpu.CompilerParams(needs_layout_passes=False),
    )(_sc_body)
    out6 = f(x6, posf)
    out_sc = (
        out6.reshape(K2 // 8, b // 128, 8, 128).transpose(0, 2, 1, 3).reshape(K2, b)
    )

    out_xt = jnp.concatenate([out_tc, out_sc], axis=0)
    return jnp.transpose(out_xt.reshape(s, d, b), (2, 0, 1))


# speedup vs baseline: 1.5063x; 1.3598x over previous
import functools
import jax
import jax.numpy as jnp
from jax import lax
from jax.experimental import pallas as pl
from jax.experimental.pallas import tpu as pltpu
from jax.experimental.pallas import tpu_sc as plsc

BATCH, SEQ, DIM = 4096, 200, 64
K = SEQ * DIM
NW = 32
NCHUNK = 6400
CPW = NCHUNK // NW
KPW = K // NW
NBUF = 5
NGRP = CPW // NBUF


def _sc_body(x_hbm, pos_hbm, out_hbm, pv, ibufs, obufs, *sems):
    gsem = sems[:NBUF]
    ssem = sems[NBUF:]
    wid = lax.axis_index("s") * 2 + lax.axis_index("c")
    base_m = wid * CPW
    base_k = wid * KPW

    pltpu.sync_copy(pos_hbm.at[pl.ds(base_k, KPW)], pv)

    def gather(m, p):
        return pltpu.make_async_copy(x_hbm.at[m], ibufs.at[p], gsem[p])

    def scatter(m, p):
        return pltpu.make_async_copy(obufs.at[p], out_hbm.at[m], ssem[p])

    for p in range(NBUF):
        gather(base_m + p, p).start()

    def gbody(g, carry):
        for p in range(NBUF):
            m = base_m + g * NBUF + p
            gather(m, p).wait()

            @pl.when(g > 0)
            def _(m=m, p=p):
                scatter(m - NBUF, p).wait()

            k0 = 8 * lax.shift_right_logical(m, 2) - base_k
            ib = ibufs.at[p]
            ob = obufs.at[p]
            for i in range(8):
                idx = jnp.full((16,), k0 + i, dtype=jnp.int32)
                splat = plsc.load_gather(pv, [idx])
                for c in range(8):
                    for t in range(8):
                        sl = pl.ds(t * 16, 16)
                        ob[c, i, sl] = ib[c, i, sl] + splat
            scatter(m, p).start()

            @pl.when(g < NGRP - 1)
            def _(m=m, p=p):
                gather(m + NBUF, p).start()

        return carry

    lax.fori_loop(0, NGRP, gbody, 0)

    for p in range(NBUF):
        scatter(base_m + (NGRP - 1) * NBUF + p, p).wait()


def kernel(x, pos_table):
    b, s, d = x.shape
    k = s * d
    xt = jnp.transpose(x, (1, 2, 0)).reshape(k, b)
    x4 = xt.reshape(k // 8, 8, b // 128, 128).transpose(0, 2, 1, 3)
    x6 = x4.reshape(NCHUNK, 8, 8, 128)
    posf = pos_table.reshape(k)

    mesh = plsc.VectorSubcoreMesh(core_axis_name="c", subcore_axis_name="s")
    f = functools.partial(
        pl.kernel,
        mesh=mesh,
        out_type=jax.ShapeDtypeStruct((NCHUNK, 8, 8, 128), jnp.float32),
        scratch_types=[
            pltpu.VMEM((KPW,), jnp.float32),
            pltpu.VMEM((NBUF, 8, 8, 128), jnp.float32),
            pltpu.VMEM((NBUF, 8, 8, 128), jnp.float32),
        ]
        + [pltpu.SemaphoreType.DMA] * (2 * NBUF),
        compiler_params=pltpu.CompilerParams(needs_layout_passes=False),
    )(_sc_body)
    out6 = f(x6, posf)

    out_xt = out6.reshape(k // 8, b // 128, 8, 128).transpose(0, 2, 1, 3).reshape(k, b)
    return jnp.transpose(out_xt.reshape(s, d, b), (2, 0, 1))
